# Initial kernel scaffold; baseline (speedup 1.0000x reference)
import jax
import jax.numpy as jnp
from jax.experimental import pallas as pl


def kernel(entity_ids, embed_weight):
    raise NotImplementedError("write your pallas kernel here")


# speedup vs baseline: 1.0942x; 1.0942x over previous
import functools

import jax
import jax.numpy as jnp
from jax import lax
from jax.experimental import pallas as pl
from jax.experimental.pallas import tpu as pltpu
from jax.experimental.pallas import tpu_sc as plsc

BATCH, HIST, DIM = 16384, 50, 32
B = BATCH * HIST
NC, NS = 2, 16
NW = NC * NS
B_PER_W = B // NW
GCHUNK = 128
STEP = 1024
NG = STEP // GCHUNK
N_STEPS = B_PER_W // STEP
ROWS_PER_W = B_PER_W // GCHUNK

@functools.lru_cache(maxsize=None)
def _build_sc_gather():
    mesh = plsc.VectorSubcoreMesh(core_axis_name="c", subcore_axis_name="s")

    @functools.partial(
        pl.kernel,
        mesh=mesh,
        compiler_params=pltpu.CompilerParams(use_tc_tiling_on_sc=False),
        out_type=jax.ShapeDtypeStruct((B, DIM), jnp.float32),
        scratch_types=[
            pltpu.VMEM((NG, GCHUNK), jnp.int32),
            pltpu.VMEM((STEP, DIM), jnp.float32),
            pltpu.SemaphoreType.DMA,
        ],
    )
    def _sc_gather(idx_hbm, table_hbm, out_hbm, idx_v, rows_v, sem):
        wid = lax.axis_index("s") * NC + lax.axis_index("c")
        base = wid * B_PER_W
        base_row = wid * ROWS_PER_W

        def step(i, carry):
            off = base + i * STEP
            pltpu.sync_copy(idx_hbm.at[pl.ds(base_row + i * NG, NG)], idx_v)
            copies = [
                pltpu.async_copy(
                    table_hbm.at[idx_v.at[j]],
                    rows_v.at[pl.ds(j * GCHUNK, GCHUNK)],
                    sem,
                )
                for j in range(NG)
            ]
            for c in copies:
                c.wait()
            pltpu.sync_copy(rows_v, out_hbm.at[pl.ds(off, STEP)])
            return carry

        lax.fori_loop(0, N_STEPS, step, 0)

    return _sc_gather


def kernel(entity_ids, embed_weight):
    idx = entity_ids.astype(jnp.int32).reshape(B // GCHUNK, GCHUNK)
    out = _build_sc_gather()(idx, embed_weight)
    return out.reshape(BATCH, HIST, DIM)


# speedup vs baseline: 1.3042x; 1.1920x over previous
import functools

import jax
import jax.numpy as jnp
from jax import lax
from jax.experimental import pallas as pl
from jax.experimental.pallas import tpu as pltpu
from jax.experimental.pallas import tpu_sc as plsc

BATCH, HIST, DIM = 16384, 50, 32
B = BATCH * HIST
NC, NS = 2, 16
NW = NC * NS
B_PER_W = B // NW
GCHUNK = 128
STEP = 1280
NG = STEP // GCHUNK
N_STEPS = B_PER_W // STEP
ROWS_PER_W = B_PER_W // GCHUNK


@functools.lru_cache(maxsize=None)
def _build_sc_gather():
    mesh = plsc.VectorSubcoreMesh(core_axis_name="c", subcore_axis_name="s")

    @functools.partial(
        pl.kernel,
        mesh=mesh,
        compiler_params=pltpu.CompilerParams(use_tc_tiling_on_sc=False),
        out_type=jax.ShapeDtypeStruct((B // GCHUNK, GCHUNK, DIM), jnp.float32),
        scratch_types=[
            pltpu.VMEM((2, NG, GCHUNK), jnp.int32),
            pltpu.VMEM((2, NG, GCHUNK, DIM), jnp.float32),
            pltpu.SemaphoreType.DMA,
            pltpu.SemaphoreType.DMA,
        ],
    )
    def _sc_gather(idx_hbm, table_hbm, out_hbm, idx_v, rows_v, sem_g, sem_o):
        wid = lax.axis_index("s") * NC + lax.axis_index("c")
        base_row = wid * ROWS_PER_W

        def load_fire(i, b):
            pltpu.sync_copy(idx_hbm.at[pl.ds(base_row + i * NG, NG)],
                            idx_v.at[b])
            for j in range(NG):
                pltpu.async_copy(table_hbm.at[idx_v.at[b].at[j]],
                                 rows_v.at[b].at[j], sem_g)

        def wait_gather(b):
            for j in range(NG):
                pltpu.make_async_copy(table_hbm.at[idx_v.at[b].at[j]],
                                      rows_v.at[b].at[j], sem_g).wait()

        def fire_out(i, b):
            pltpu.async_copy(rows_v.at[b],
                             out_hbm.at[pl.ds(base_row + i * NG, NG)], sem_o)

        def wait_out(i, b):
            pltpu.make_async_copy(rows_v.at[b],
                                  out_hbm.at[pl.ds(base_row + i * NG, NG)],
                                  sem_o).wait()

        load_fire(0, 0)

        def pair(p, carry):
            for b in (0, 1):
                i = 2 * p + b
                wait_gather(b)
                fire_out(i, b)
                nb = 1 - b

                @pl.when(i + 1 < N_STEPS)
                def _prefetch():
                    @pl.when(i > 0)
                    def _free_buf():
                        wait_out(i - 1, nb)

                    load_fire(i + 1, nb)

            return carry

        lax.fori_loop(0, N_STEPS // 2, pair, 0)
        wait_out(N_STEPS - 2, 0)
        wait_out(N_STEPS - 1, 1)

    return _sc_gather


def kernel(entity_ids, embed_weight):
    idx = entity_ids.astype(jnp.int32).reshape(B // GCHUNK, GCHUNK)
    out = _build_sc_gather()(idx, embed_weight)
    return out.reshape(BATCH, HIST, DIM)


# speedup vs baseline: 1.4628x; 1.1216x over previous
import functools

import jax
import jax.numpy as jnp
from jax import lax
from jax.experimental import pallas as pl
from jax.experimental.pallas import tpu as pltpu
from jax.experimental.pallas import tpu_sc as plsc

BATCH, HIST, DIM = 16384, 50, 32
NC, NS, L = 2, 16, 16
NW = NC * NS
B_PER_W = BATCH // NW
BSUB = 16
N_SUB = B_PER_W // BSUB
HPAD = 56
DT, DI = DIM // 8, 8
BT, BI = BATCH // 128, 128


@functools.lru_cache(maxsize=None)
def _build_sc_gather():
    mesh = plsc.VectorSubcoreMesh(core_axis_name="c", subcore_axis_name="s")

    @functools.partial(
        pl.kernel,
        mesh=mesh,
        compiler_params=pltpu.CompilerParams(use_tc_tiling_on_sc=False,
                                             needs_layout_passes=False),
        out_type=jax.ShapeDtypeStruct((HPAD, DT, BT, DI, BI), jnp.float32),
        scratch_types=[
            pltpu.VMEM((2, HIST, BSUB), jnp.int32),
            pltpu.VMEM((2, HIST * BSUB), jnp.int32),
            pltpu.VMEM((2, HIST * BSUB, DIM), jnp.float32),
            pltpu.VMEM((2, HIST, DT, DI, BSUB), jnp.float32),
            pltpu.SemaphoreType.DMA,
            pltpu.SemaphoreType.DMA,
        ],
    )
    def _sc_gather(idx_hbm, table_hbm, out_hbm, idx_v, idxf_v, gath_v,
                   stage_v, sem_g, sem_o):
        wid = lax.axis_index("s") * NC + lax.axis_index("c")
        b_w = wid * B_PER_W
        chunks = [(j * 128, 128) for j in range(HIST * BSUB // 128)]
        if HIST * BSUB % 128:
            chunks.append((HIST * BSUB - HIST * BSUB % 128,
                           HIST * BSUB % 128))

        def gather_copies(b):
            return [
                pltpu.make_async_copy(
                    table_hbm.at[idxf_v.at[b].at[pl.ds(off, n)]],
                    gath_v.at[b].at[pl.ds(off, n)], sem_g)
                for off, n in chunks
            ]

        def load_fire(k, b):
            b0 = b_w + k * BSUB
            pltpu.sync_copy(idx_hbm.at[:, pl.ds(b0, BSUB)], idx_v.at[b])
            for h in range(HIST):
                idxf_v[b, pl.ds(h * BSUB, BSUB)] = idx_v[b, h, :]
            for c in gather_copies(b):
                c.start()

        def wait_gather(b):
            for c in gather_copies(b):
                c.wait()

        def out_dst(k):
            b0 = b_w + k * BSUB
            bt = b0 // BI
            bi0 = b0 % BI
            return out_hbm.at[pl.ds(0, HIST), :, bt, :, pl.ds(bi0, BSUB)]

        def fire_out(k, b):
            pltpu.async_copy(stage_v.at[b], out_dst(k), sem_o)

        def wait_out(k, b):
            pltpu.make_async_copy(stage_v.at[b], out_dst(k), sem_o).wait()

        lanes = lax.iota(jnp.int32, L)

        def transpose(b):
            gref = gath_v.at[b]
            sref = stage_v.at[b]

            def h_body(h, carry):
                rows = h * BSUB + lanes
                for d in range(DIM):
                    dv = jnp.full((L,), d, jnp.int32)
                    v = plsc.load_gather(gref, [rows, dv])
                    sref[h, d // DI, d % DI, :] = v
                return carry

            lax.fori_loop(0, HIST, h_body, 0)

        load_fire(0, 0)

        def step(p, carry):
            for b in (0, 1):
                k = 2 * p + b
                wait_gather(b)

                @pl.when(k + 1 < N_SUB)
                def _prefetch():
                    load_fire(k + 1, 1 - b)

                @pl.when(k >= 2)
                def _free_stage():
                    wait_out(k - 2, b)

                transpose(b)
                fire_out(k, b)
            return carry

        lax.fori_loop(0, N_SUB // 2, step, 0)
        wait_out(N_SUB - 2, 0)
        wait_out(N_SUB - 1, 1)

    return _sc_gather


def kernel(entity_ids, embed_weight):
    ids_t = entity_ids.astype(jnp.int32).T
    out6 = _build_sc_gather()(ids_t, embed_weight)
    out = out6.transpose(2, 4, 0, 1, 3).reshape(BATCH, HPAD, DIM)
    return out[:, :HIST, :]


# speedup vs baseline: 2.2006x; 1.5044x over previous
import functools

import jax
import jax.numpy as jnp
from jax import lax
from jax.experimental import pallas as pl
from jax.experimental.pallas import tpu as pltpu
from jax.experimental.pallas import tpu_sc as plsc

BATCH, HIST, DIM = 16384, 50, 32
NC, NS, L = 2, 16, 16
NW = NC * NS
B_PER_W = BATCH // NW
BSUB = 16
N_SUB = B_PER_W // BSUB
HPAD = 56
DT, DI = DIM // 8, 8
BT, BI = BATCH // 128, 128


@functools.lru_cache(maxsize=None)
def _build_sc_gather():
    mesh = plsc.VectorSubcoreMesh(core_axis_name="c", subcore_axis_name="s")

    @functools.partial(
        pl.kernel,
        mesh=mesh,
        compiler_params=pltpu.CompilerParams(use_tc_tiling_on_sc=False,
                                             needs_layout_passes=False),
        out_type=jax.ShapeDtypeStruct((HPAD, DT, BT, DI, BI), jnp.float32),
        scratch_types=[
            pltpu.VMEM((HIST, BSUB), jnp.int32),
            pltpu.VMEM((HIST * BSUB,), jnp.int32),
            pltpu.VMEM((2, HIST * BSUB, DIM), jnp.float32),
            pltpu.VMEM((2, HIST, DT, DI, BSUB + 1), jnp.float32),
            pltpu.SemaphoreType.DMA,
            pltpu.SemaphoreType.DMA,
        ],
    )
    def _sc_gather(idx_hbm, table_hbm, out_hbm, idx_v, idxf_v, gath_v,
                   stage_v, sem_g, sem_o):
        wid = lax.axis_index("s") * NC + lax.axis_index("c")
        b_w = wid * B_PER_W
        chunks = [(j * 128, 128) for j in range(HIST * BSUB // 128)]
        if HIST * BSUB % 128:
            chunks.append((HIST * BSUB - HIST * BSUB % 128,
                           HIST * BSUB % 128))

        def gather_copies(b):
            return [
                pltpu.make_async_copy(
                    table_hbm.at[idxf_v.at[pl.ds(off, n)]],
                    gath_v.at[b].at[pl.ds(off, n)], sem_g)
                for off, n in chunks
            ]

        def load_fire(k, b):
            b0 = b_w + k * BSUB
            pltpu.sync_copy(idx_hbm.at[:, pl.ds(b0, BSUB)], idx_v)
            for h in range(HIST):
                idxf_v[pl.ds(h * BSUB, BSUB)] = idx_v[h, :]
            for c in gather_copies(b):
                c.start()

        def wait_gather(b):
            for c in gather_copies(b):
                c.wait()

        def out_src(b):
            return stage_v.at[b].at[:, :, :, pl.ds(0, BSUB)]

        def out_dst(k):
            b0 = b_w + k * BSUB
            bt = b0 // BI
            bi0 = b0 % BI
            return out_hbm.at[pl.ds(0, HIST), :, bt, :, pl.ds(bi0, BSUB)]

        def fire_out(k, b):
            pltpu.async_copy(out_src(b), out_dst(k), sem_o)

        def wait_out(k, b):
            pltpu.make_async_copy(out_src(b), out_dst(k), sem_o).wait()

        lanes = lax.iota(jnp.int32, L)
        dt_vecs = [(d0 + lanes) // DI for d0 in (0, L)]
        di_vecs = [(d0 + lanes) % DI for d0 in (0, L)]
        bb_vecs = [jnp.full((L,), bb, jnp.int32) for bb in range(BSUB)]

        def transpose(b):
            gref = gath_v.at[b]
            sref = stage_v.at[b]

            HU = 10

            def h_body(hb, carry):
                h0 = hb * HU
                for hh in range(HU):
                    h = h0 + hh
                    hv = jnp.full((L,), h, jnp.int32)
                    for bb in range(BSUB):
                        r = h * BSUB + bb
                        bv = bb_vecs[bb]
                        for half in range(2):
                            v = gref[r, pl.ds(half * L, L)]
                            plsc.store_scatter(
                                sref,
                                [hv, dt_vecs[half], di_vecs[half], bv], v)
                return carry

            lax.fori_loop(0, HIST // HU, h_body, 0)

        load_fire(0, 0)

        def step(p, carry):
            for b in (0, 1):
                k = 2 * p + b
                wait_gather(b)

                @pl.when(k + 1 < N_SUB)
                def _prefetch():
                    load_fire(k + 1, 1 - b)

                @pl.when(k >= 2)
                def _free_stage():
                    wait_out(k - 2, b)

                transpose(b)
                fire_out(k, b)
            return carry

        lax.fori_loop(0, N_SUB // 2, step, 0)
        wait_out(N_SUB - 2, 0)
        wait_out(N_SUB - 1, 1)

    return _sc_gather


def kernel(entity_ids, embed_weight):
    ids_t = entity_ids.astype(jnp.int32).T
    out6 = _build_sc_gather()(ids_t, embed_weight)
    out = out6.transpose(2, 4, 0, 1, 3).reshape(BATCH, HPAD, DIM)
    return out[:, :HIST, :]
